# Initial kernel scaffold; baseline (speedup 1.0000x reference)
import jax
import jax.numpy as jnp
from jax.experimental import pallas as pl


def kernel(output, target):
    raise NotImplementedError("write your pallas kernel here")


# speedup vs baseline: 87.7571x; 87.7571x over previous
import jax
import jax.numpy as jnp
from jax.experimental import pallas as pl

_EPS = 0.1


def _row_stats_kernel(x_ref, tgt_ref, loss_ref, margin_ref):
    x = x_ref[...]
    tgt = tgt_ref[...]
    v = x.shape[1]
    idx = jax.lax.broadcasted_iota(jnp.int32, x.shape, 1)
    m1 = jnp.max(x, axis=1, keepdims=True)
    fi = jnp.min(jnp.where(x == m1, idx, v), axis=1, keepdims=True)
    m2 = jnp.max(jnp.where(idx == fi, -jnp.inf, x), axis=1, keepdims=True)
    out_t = jnp.sum(jnp.where(idx == tgt, x, 0.0), axis=1, keepdims=True)
    lse = m1 + jnp.log(jnp.sum(jnp.exp(x - m1), axis=1, keepdims=True))
    margin1 = out_t - m1
    margin = jnp.where(margin1 != 0.0, margin1, out_t - m2)
    fst = jax.nn.relu(1.0 - margin)
    snd = jax.nn.relu(1.0 - out_t + lse)
    loss_ref[...] = jnp.where(margin >= 0.0, fst, snd)
    margin_ref[...] = margin


def _select_kernel(lc_ref, lr_ref, margin_ref, out_ref):
    b = lc_ref.shape[0]
    lc = lc_ref[...]
    lr = lr_ref[...]
    margin = margin_ref[...]
    neg = jnp.sum((margin < 0.0).astype(jnp.float32))
    threshold = (1.0 - _EPS) ** 2 * b + (1.0 - _EPS) * neg
    ii = jax.lax.broadcasted_iota(jnp.int32, (b, b), 0)
    jj = jax.lax.broadcasted_iota(jnp.int32, (b, b), 1)
    before = ((lr < lc) | ((lr == lc) & (jj <= ii))).astype(jnp.float32)
    rank = jnp.sum(before, axis=1, keepdims=True) - 1.0
    psum = jnp.sum(before * lr, axis=1, keepdims=True)
    sel = (psum <= threshold + 1.0 - rank).astype(jnp.float32)
    npcl1 = jnp.sum(lc * sel)
    cnt = jnp.sum(sel)
    npcl2 = threshold - cnt
    out_ref[...] = jnp.full((1, 1), jnp.maximum(npcl1, npcl2) / cnt, jnp.float32)


def kernel(output, target):
    b, v = output.shape
    rb = 16
    tgt2d = target.astype(jnp.int32).reshape(b, 1)
    loss, margin = pl.pallas_call(
        _row_stats_kernel,
        grid=(b // rb,),
        in_specs=[
            pl.BlockSpec((rb, v), lambda i: (i, 0)),
            pl.BlockSpec((rb, 1), lambda i: (i, 0)),
        ],
        out_specs=[
            pl.BlockSpec((rb, 1), lambda i: (i, 0)),
            pl.BlockSpec((rb, 1), lambda i: (i, 0)),
        ],
        out_shape=[
            jax.ShapeDtypeStruct((b, 1), jnp.float32),
            jax.ShapeDtypeStruct((b, 1), jnp.float32),
        ],
    )(output, tgt2d)
    loss_r = loss.reshape(1, b)
    res = pl.pallas_call(
        _select_kernel,
        out_shape=jax.ShapeDtypeStruct((1, 1), jnp.float32),
    )(loss, loss_r, margin)
    return res[0, 0]


# speedup vs baseline: 97.8113x; 1.1146x over previous
import jax
import jax.numpy as jnp
from jax.experimental import pallas as pl

_EPS = 0.1


def _row_stats_kernel(x_ref, tgt_ref, loss_ref, margin_ref):
    x = x_ref[...]
    tgt = tgt_ref[...]
    idx = jax.lax.broadcasted_iota(jnp.int32, x.shape, 1)
    eqt = idx == tgt
    m1 = jnp.max(x, axis=1, keepdims=True)
    out_t = jnp.sum(jnp.where(eqt, x, 0.0), axis=1, keepdims=True)
    m2t = jnp.max(jnp.where(eqt, -jnp.inf, x), axis=1, keepdims=True)
    lse = m1 + jnp.log(jnp.sum(jnp.exp(x - m1), axis=1, keepdims=True))
    margin = jnp.where(out_t == m1, out_t - m2t, out_t - m1)
    fst = jax.nn.relu(1.0 - margin)
    snd = jax.nn.relu(1.0 - out_t + lse)
    loss_ref[...] = jnp.where(margin >= 0.0, fst, snd)
    margin_ref[...] = margin


def _select_kernel(lc_ref, lr_ref, margin_ref, out_ref):
    b = lc_ref.shape[0]
    lc = lc_ref[...]
    lr = lr_ref[...]
    margin = margin_ref[...]
    neg = jnp.sum((margin < 0.0).astype(jnp.float32))
    threshold = (1.0 - _EPS) ** 2 * b + (1.0 - _EPS) * neg
    ii = jax.lax.broadcasted_iota(jnp.int32, (b, b), 0)
    jj = jax.lax.broadcasted_iota(jnp.int32, (b, b), 1)
    before = ((lr < lc) | ((lr == lc) & (jj <= ii))).astype(jnp.float32)
    rank = jnp.sum(before, axis=1, keepdims=True) - 1.0
    psum = jnp.sum(before * lr, axis=1, keepdims=True)
    sel = (psum <= threshold + 1.0 - rank).astype(jnp.float32)
    npcl1 = jnp.sum(lc * sel)
    cnt = jnp.sum(sel)
    npcl2 = threshold - cnt
    out_ref[...] = jnp.full((1, 1), jnp.maximum(npcl1, npcl2) / cnt, jnp.float32)


def kernel(output, target):
    b, v = output.shape
    rb = 16
    tgt2d = target.astype(jnp.int32).reshape(b, 1)
    loss, margin = pl.pallas_call(
        _row_stats_kernel,
        grid=(b // rb,),
        in_specs=[
            pl.BlockSpec((rb, v), lambda i: (i, 0)),
            pl.BlockSpec((rb, 1), lambda i: (i, 0)),
        ],
        out_specs=[
            pl.BlockSpec((rb, 1), lambda i: (i, 0)),
            pl.BlockSpec((rb, 1), lambda i: (i, 0)),
        ],
        out_shape=[
            jax.ShapeDtypeStruct((b, 1), jnp.float32),
            jax.ShapeDtypeStruct((b, 1), jnp.float32),
        ],
    )(output, tgt2d)
    loss_r = loss.reshape(1, b)
    res = pl.pallas_call(
        _select_kernel,
        out_shape=jax.ShapeDtypeStruct((1, 1), jnp.float32),
    )(loss, loss_r, margin)
    return res[0, 0]
